# baseline (speedup 1.0000x reference)
import functools

import jax
import jax.numpy as jnp
from jax.experimental import pallas as pl
from jax.experimental.pallas import tpu as pltpu

_N_FRAMES = 1000
_C, _H, _W = 1, 192, 256
_B = 64
_G = 64


def _tc_body(idx_ref, *refs):
    in_refs = refs[:_G]
    out_ref = refs[_G]
    for k in range(_G):
        x = in_refs[k][...]
        out_ref[k, :, :] = jnp.maximum(x[0], 0.0) + jnp.exp(jnp.minimum(x[0], 0.0))


def _mk_spec(k):
    return pl.BlockSpec((1, _H, _W), lambda i, idx, k=k: (idx[i * _G + k], 0, 0))


def kernel(indices, maps):
    idx = indices.astype(jnp.int32)
    table = maps.reshape(_N_FRAMES, _H, _W)
    out = pl.pallas_call(
        _tc_body,
        grid_spec=pltpu.PrefetchScalarGridSpec(
            num_scalar_prefetch=1,
            grid=(_B // _G,),
            in_specs=[_mk_spec(k) for k in range(_G)],
            out_specs=pl.BlockSpec((_G, _H, _W), lambda i, idx: (i, 0, 0)),
        ),
        out_shape=jax.ShapeDtypeStruct((_B, _H, _W), jnp.float32),
    )(idx, *([table] * _G))
    return out.reshape(_B, _C, _H, _W)


# speedup vs baseline: 1.0872x; 1.0872x over previous
import functools

import jax
import jax.numpy as jnp
from jax.experimental import pallas as pl
from jax.experimental.pallas import tpu as pltpu

_N_FRAMES = 1000
_C, _H, _W = 1, 192, 256
_B = 64
_G = 32
_HB = 96


def _tc_body(idx_ref, *refs):
    in_refs = refs[:_G]
    out_ref = refs[_G]
    for k in range(_G):
        x = in_refs[k][...]
        out_ref[k, :, :] = jnp.maximum(x[0], 0.0) + jnp.exp(jnp.minimum(x[0], 0.0))


def _mk_spec(k):
    return pl.BlockSpec(
        (1, _HB, _W), lambda i, j, idx, k=k: (idx[i * _G + k], j, 0)
    )


def kernel(indices, maps):
    idx = indices.astype(jnp.int32)
    table = maps.reshape(_N_FRAMES, _H, _W)
    out = pl.pallas_call(
        _tc_body,
        grid_spec=pltpu.PrefetchScalarGridSpec(
            num_scalar_prefetch=1,
            grid=(_B // _G, _H // _HB),
            in_specs=[_mk_spec(k) for k in range(_G)],
            out_specs=pl.BlockSpec((_G, _HB, _W), lambda i, j, idx: (i, j, 0)),
        ),
        out_shape=jax.ShapeDtypeStruct((_B, _H, _W), jnp.float32),
    )(idx, *([table] * _G))
    return out.reshape(_B, _C, _H, _W)


# speedup vs baseline: 1.2644x; 1.1630x over previous
import jax
import jax.numpy as jnp
from jax.experimental import pallas as pl
from jax.experimental.pallas import tpu as pltpu

_N_FRAMES = 1000
_C, _H, _W = 1, 192, 256
_B = 64
_G = 32


def _gather_elu_body(idx_ref, *refs):
    in_refs = refs[:_G]
    out_ref = refs[_G]
    for k in range(_G):
        x = in_refs[k][...]
        out_ref[k, :, :] = jnp.maximum(x[0], 0.0) + jnp.exp(jnp.minimum(x[0], 0.0))


def _mk_in_spec(k):
    return pl.BlockSpec((1, _H, _W), lambda i, idx, k=k: (idx[i * _G + k], 0, 0))


def kernel(indices, maps):
    idx = indices.astype(jnp.int32)
    table = maps.reshape(_N_FRAMES, _H, _W)
    out = pl.pallas_call(
        _gather_elu_body,
        grid_spec=pltpu.PrefetchScalarGridSpec(
            num_scalar_prefetch=1,
            grid=(_B // _G,),
            in_specs=[_mk_in_spec(k) for k in range(_G)],
            out_specs=pl.BlockSpec((_G, _H, _W), lambda i, idx: (i, 0, 0)),
        ),
        out_shape=jax.ShapeDtypeStruct((_B, _H, _W), jnp.float32),
    )(idx, *([table] * _G))
    return out.reshape(_B, _C, _H, _W)
